# baseline (speedup 1.0000x reference)
import functools
import math

import jax
import jax.numpy as jnp
from jax import lax
from jax.experimental import pallas as pl
from jax.experimental.pallas import tpu as pltpu
from jax.experimental.pallas import tpu_sc as plsc

F32 = jnp.float32
D_MODEL = 256
N_HEADS = 8
DH = 32
KNN = 16
B = 8
N_OBJ = 48
T = 11
N_MAP = 512
P_MAP = 20
N_TOK = N_OBJ * T + N_MAP
N_PAD = 1152
T_PAD = 16
P_PAD = 24
NEG = -1e30
BIG = 1e30


def _relu(x):
    return jnp.maximum(x, 0.0)


def _mm(a, b):
    return jnp.dot(a.astype(jnp.bfloat16), b.astype(jnp.bfloat16),
                   preferred_element_type=F32)


def _ln(x, g, b):
    mu = jnp.mean(x, axis=1, keepdims=True)
    xc = x - mu
    var = jnp.mean(xc * xc, axis=1, keepdims=True)
    return xc * lax.rsqrt(var + 1e-5) * g + b


def _obj_pn_body(xin, w0, b0, w1, b1, w2, b2, w3, b3, w4, b4, out):
    x = xin[...]
    h = _relu(jnp.dot(x, w0[...], preferred_element_type=F32) + b0[...])
    rid = lax.broadcasted_iota(jnp.int32, h.shape, 0)
    t = rid & (T_PAD - 1)
    hm = jnp.where(t < T, h, 0.0)
    pooled = jnp.max(hm.reshape(B * N_OBJ, T_PAD, D_MODEL), axis=1, keepdims=True)
    pooledb = jnp.broadcast_to(pooled, (B * N_OBJ, T_PAD, D_MODEL))
    h2 = jnp.concatenate([h, pooledb.reshape(h.shape)], axis=1)
    h2 = _relu(jnp.dot(h2, w1[...], preferred_element_type=F32) + b1[...])
    h2 = _relu(jnp.dot(h2, w2[...], preferred_element_type=F32) + b2[...])
    h3 = _relu(jnp.dot(h2, w3[...], preferred_element_type=F32) + b3[...])
    out[...] = jnp.dot(h3, w4[...], preferred_element_type=F32) + b4[...]


def _map_pn_body(xin, w0, b0, w1, b1, w2, b2, wm0, bm0, wm1, bm1,
                 wo0, bo0, wo1, bo1, out):
    x = xin[0]
    h = _relu(jnp.dot(x, w0[...], preferred_element_type=F32) + b0[...])
    h = _relu(jnp.dot(h, w1[...], preferred_element_type=F32) + b1[...])
    h = _relu(jnp.dot(h, w2[...], preferred_element_type=F32) + b2[...])
    rid = lax.broadcasted_iota(jnp.int32, h.shape, 0)
    t = jnp.remainder(rid, P_PAD)
    hm = jnp.where(t < P_MAP, h, 0.0)
    pooled = jnp.max(hm.reshape(N_MAP, P_PAD, 64), axis=1, keepdims=True)
    pooledb = jnp.broadcast_to(pooled, (N_MAP, P_PAD, 64))
    h2 = jnp.concatenate([h, pooledb.reshape(h.shape)], axis=1)
    h2 = _relu(jnp.dot(h2, wm0[...], preferred_element_type=F32) + bm0[...])
    h2 = _relu(jnp.dot(h2, wm1[...], preferred_element_type=F32) + bm1[...])
    h2m = jnp.where(t < P_MAP, h2, 0.0)
    pooled2 = jnp.max(h2m.reshape(N_MAP, P_PAD, 64), axis=1)
    o = _relu(jnp.dot(pooled2, wo0[...], preferred_element_type=F32) + bo0[...])
    out[0] = jnp.dot(o, wo1[...], preferred_element_type=F32) + bo1[...]


def _prep_body(pos, posT, bias_ref, pe_ref):
    p = pos[0]
    pt = posT[0]
    xc = p[:, 0:1]
    yc = p[:, 1:2]
    zc = p[:, 2:3]
    dx = xc - pt[0:1, :]
    dy = yc - pt[1:2, :]
    dz = zc - pt[2:3, :]
    bias_ref[0] = dx * dx + dy * dy + dz * dz
    c = lax.broadcasted_iota(jnp.int32, (N_TOK, D_MODEL), 1)
    cc = c & 127
    j = (cc >> 1).astype(F32)
    invf = jnp.exp(j * (-math.log(10000.0) / 64.0))
    part = jnp.where(c < 128, yc, xc)
    val = part * (2.0 * math.pi) * invf
    pe_ref[0] = jnp.where((cc & 1) == 0, jnp.sin(val), jnp.cos(val))


_SC_WORKERS = 32
_SC_ROWS = (B * N_TOK) // _SC_WORKERS
_SC_CHUNKS = N_TOK // 16


def _sc_topk_body(d2_hbm, mask_hbm, row_v, mask_v):
    wid = lax.axis_index("s") * 2 + lax.axis_index("c")
    base = wid * _SC_ROWS
    zeros16 = jnp.zeros((16,), F32)
    ones16 = jnp.ones((16,), F32)

    def zero_body(i, carry):
        mask_v[pl.ds(i * 16, 16)] = zeros16
        return carry

    lax.fori_loop(0, _SC_CHUNKS, zero_body, 0)

    def row_body(r, carry):
        pltpu.sync_copy(d2_hbm.at[base + r], row_v)

        def chunk_body(c, cur):
            cur_d, cur_i = cur
            chunk = row_v[pl.ds(c * 16, 16)]
            idxv = lax.iota(jnp.int32, 16) + c * 16
            sd, si = plsc.sort_key_val(chunk, idxv)
            rd = lax.rev(cur_d, (0,))
            ri = lax.rev(cur_i, (0,))
            take = rd <= sd
            md = jnp.where(take, rd, sd)
            mi = jnp.where(take, ri, si)
            out = plsc.sort_key_val(md, mi)
            return (out[0], out[1])

        init = (jnp.full((16,), BIG, F32), jnp.zeros((16,), jnp.int32))
        _, cur_i = lax.fori_loop(0, _SC_CHUNKS, chunk_body, init)
        plsc.store_scatter(mask_v, [cur_i], ones16)
        pltpu.sync_copy(mask_v, mask_hbm.at[base + r])
        plsc.store_scatter(mask_v, [cur_i], zeros16)
        return carry

    lax.fori_loop(0, _SC_ROWS, row_body, 0)


def _sc_topk_mask(d2_flat):
    mesh = plsc.VectorSubcoreMesh(core_axis_name="c", subcore_axis_name="s")
    fn = functools.partial(
        pl.kernel,
        out_type=jax.ShapeDtypeStruct((B * N_TOK, N_TOK), F32),
        mesh=mesh,
        scratch_types=[pltpu.VMEM((N_TOK,), F32),
                       pltpu.VMEM((N_TOK,), F32)],
        compiler_params=pltpu.CompilerParams(needs_layout_passes=False),
    )(_sc_topk_body)
    return fn(d2_flat)


def _layer_body(x_ref, pe_ref, bias_ref, wq, bq, wk, bk, wv, bv, wo, bo,
                w1, bf1, w2, bf2, g1, be1, g2, be2, out_ref):
    x = x_ref[0]
    pe = pe_ref[0]
    mask01 = bias_ref[0].astype(jnp.bfloat16)
    qk = x + pe
    q = (_mm(qk, wq[...]) + bq[...]) * (DH ** -0.5)
    k = (_mm(qk, wk[...]) + bk[...]).astype(jnp.bfloat16)
    v = (_mm(x, wv[...]) + bv[...]).astype(jnp.bfloat16)
    qb = q.astype(jnp.bfloat16)
    ones8 = jnp.ones((N_TOK, 8), jnp.bfloat16)
    parts = []
    for h in range(N_HEADS):
        s = slice(DH * h, DH * (h + 1))
        S = lax.dot_general(qb[:, s], k[:, s], (((1,), (1,)), ((), ())),
                            preferred_element_type=F32)
        Eb = (jnp.exp(jnp.clip(S, -80.0, 60.0)) * mask01).astype(jnp.bfloat16)
        va = jnp.concatenate([v[:, s], ones8], axis=1)
        pv = jnp.dot(Eb, va, preferred_element_type=F32)
        parts.append(pv[:, :DH] / pv[:, DH:DH + 1])
    attn = jnp.concatenate(parts, axis=1)
    src = _ln(x + _mm(attn, wo[...]) + bo[...],
              g1[...], be1[...])
    ff = _mm(_relu(_mm(src, w1[...]) + bf1[...]), w2[...]) + bf2[...]
    out_ref[0] = _ln(src + ff, g2[...], be2[...])


def _wt(l):
    return l['W'].T


def _bb(l):
    return l['b'][None, :]


def kernel(obj_trajs, obj_trajs_mask, map_polylines, map_polylines_mask,
           obj_trajs_last_pos, obj_trajs_pos, map_polylines_center, params):
    p = params

    obj_in = jnp.concatenate(
        [obj_trajs, obj_trajs_mask[..., None].astype(F32)], axis=-1)
    obj_in = obj_in.reshape(B * N_OBJ, T, 30)
    obj_in = jnp.pad(obj_in, ((0, 0), (0, T_PAD - T), (0, 2)))
    obj_in = obj_in.reshape(B * N_OBJ * T_PAD, 32)
    w_pre = jnp.pad(_wt(p['agent_pre'][0]), ((0, 2), (0, 0)))
    obj_feat = pl.pallas_call(
        _obj_pn_body,
        out_shape=jax.ShapeDtypeStruct((B * N_OBJ * T_PAD, D_MODEL), F32),
    )(obj_in, w_pre, _bb(p['agent_pre'][0]),
      _wt(p['agent_mid'][0]), _bb(p['agent_mid'][0]),
      _wt(p['agent_mid'][1]), _bb(p['agent_mid'][1]),
      _wt(p['agent_out'][0]), _bb(p['agent_out'][0]),
      _wt(p['agent_out'][1]), _bb(p['agent_out'][1]))

    map_in = jnp.pad(map_polylines, ((0, 0), (0, 0), (0, P_PAD - P_MAP), (0, 7)))
    map_in = map_in.reshape(B, N_MAP * P_PAD, 16)
    w_mpre = jnp.pad(_wt(p['map_pre'][0]), ((0, 7), (0, 0)))
    full2 = lambda shp: pl.BlockSpec(shp, lambda b: (0, 0))
    map_feat = pl.pallas_call(
        _map_pn_body,
        grid=(B,),
        in_specs=[pl.BlockSpec((1, N_MAP * P_PAD, 16), lambda b: (b, 0, 0)),
                  full2((16, 64)), full2((1, 64)),
                  full2((64, 64)), full2((1, 64)),
                  full2((64, 64)), full2((1, 64)),
                  full2((128, 64)), full2((1, 64)),
                  full2((64, 64)), full2((1, 64)),
                  full2((64, 64)), full2((1, 64)),
                  full2((64, 256)), full2((1, 256))],
        out_specs=pl.BlockSpec((1, N_MAP, D_MODEL), lambda b: (b, 0, 0)),
        out_shape=jax.ShapeDtypeStruct((B, N_MAP, D_MODEL), F32),
    )(map_in, w_mpre, _bb(p['map_pre'][0]),
      _wt(p['map_pre'][1]), _bb(p['map_pre'][1]),
      _wt(p['map_pre'][2]), _bb(p['map_pre'][2]),
      _wt(p['map_mid'][0]), _bb(p['map_mid'][0]),
      _wt(p['map_mid'][1]), _bb(p['map_mid'][1]),
      _wt(p['map_out'][0]), _bb(p['map_out'][0]),
      _wt(p['map_out'][1]), _bb(p['map_out'][1]))

    obj_seq = obj_feat.reshape(B * N_OBJ, T_PAD, D_MODEL)[:, :T]
    obj_seq = obj_seq.reshape(B, N_OBJ * T, D_MODEL)
    tok = jnp.concatenate([obj_seq, map_feat], axis=1)
    pos = jnp.concatenate(
        [obj_trajs_pos.reshape(B, N_OBJ * T, 3), map_polylines_center], axis=1)
    pos = jnp.pad(pos, ((0, 0), (0, 0), (0, 5)))
    posT = jnp.swapaxes(pos, 1, 2)

    d2, pe = pl.pallas_call(
        _prep_body,
        grid=(B,),
        in_specs=[pl.BlockSpec((1, N_TOK, 8), lambda b: (b, 0, 0)),
                  pl.BlockSpec((1, 8, N_TOK), lambda b: (b, 0, 0))],
        out_specs=[pl.BlockSpec((1, N_TOK, N_TOK), lambda b: (b, 0, 0)),
                   pl.BlockSpec((1, N_TOK, D_MODEL), lambda b: (b, 0, 0))],
        out_shape=[jax.ShapeDtypeStruct((B, N_TOK, N_TOK), F32),
                   jax.ShapeDtypeStruct((B, N_TOK, D_MODEL), F32)],
    )(pos, posT)

    bias = _sc_topk_mask(d2.reshape(B * N_TOK, N_TOK)).reshape(B, N_TOK, N_TOK)

    x = tok
    row_spec = pl.BlockSpec((1, N_TOK, D_MODEL), lambda b: (b, 0, 0))
    layer_specs = [row_spec, row_spec,
                   pl.BlockSpec((1, N_TOK, N_TOK), lambda b: (b, 0, 0)),
                   full2((D_MODEL, D_MODEL)), full2((1, D_MODEL)),
                   full2((D_MODEL, D_MODEL)), full2((1, D_MODEL)),
                   full2((D_MODEL, D_MODEL)), full2((1, D_MODEL)),
                   full2((D_MODEL, D_MODEL)), full2((1, D_MODEL)),
                   full2((D_MODEL, 4 * D_MODEL)), full2((1, 4 * D_MODEL)),
                   full2((4 * D_MODEL, D_MODEL)), full2((1, D_MODEL)),
                   full2((1, D_MODEL)), full2((1, D_MODEL)),
                   full2((1, D_MODEL)), full2((1, D_MODEL))]
    for lp in p['layers']:
        x = pl.pallas_call(
            _layer_body,
            grid=(B,),
            in_specs=layer_specs,
            out_specs=row_spec,
            out_shape=jax.ShapeDtypeStruct((B, N_TOK, D_MODEL), F32),
            compiler_params=pltpu.CompilerParams(
                dimension_semantics=("parallel",)),
        )(x, pe, bias,
          _wt(lp['q']), _bb(lp['q']), _wt(lp['k']), _bb(lp['k']),
          _wt(lp['v']), _bb(lp['v']), _wt(lp['o']), _bb(lp['o']),
          _wt(lp['ff1']), _bb(lp['ff1']), _wt(lp['ff2']), _bb(lp['ff2']),
          lp['ln1_g'][None, :], lp['ln1_b'][None, :],
          lp['ln2_g'][None, :], lp['ln2_b'][None, :])

    obj_out = x[:, :N_OBJ * T].reshape(B, N_OBJ, T, D_MODEL)
    map_out = jnp.broadcast_to(
        x[:, N_OBJ * T:][:, :, None, :], (B, N_MAP, T, D_MODEL))
    map_mask = map_polylines_mask.sum(axis=-1) > 0
    return obj_out, map_out, obj_trajs_mask, map_mask, obj_trajs_last_pos


# speedup vs baseline: 1.0007x; 1.0007x over previous
import functools
import math

import jax
import jax.numpy as jnp
from jax import lax
from jax.experimental import pallas as pl
from jax.experimental.pallas import tpu as pltpu
from jax.experimental.pallas import tpu_sc as plsc

F32 = jnp.float32
D_MODEL = 256
N_HEADS = 8
DH = 32
KNN = 16
B = 8
N_OBJ = 48
T = 11
N_MAP = 512
P_MAP = 20
N_TOK = N_OBJ * T + N_MAP
N_PAD = 1152
T_PAD = 16
P_PAD = 24
NEG = -1e30
BIG = 1e30


def _relu(x):
    return jnp.maximum(x, 0.0)


def _mm(a, b):
    return jnp.dot(a.astype(jnp.bfloat16), b.astype(jnp.bfloat16),
                   preferred_element_type=F32)


def _ln(x, g, b):
    mu = jnp.mean(x, axis=1, keepdims=True)
    xc = x - mu
    var = jnp.mean(xc * xc, axis=1, keepdims=True)
    return xc * lax.rsqrt(var + 1e-5) * g + b


def _obj_pn_body(xin, w0, b0, w1, b1, w2, b2, w3, b3, w4, b4, out):
    x = xin[...]
    h = _relu(jnp.dot(x, w0[...], preferred_element_type=F32) + b0[...])
    rid = lax.broadcasted_iota(jnp.int32, h.shape, 0)
    t = rid & (T_PAD - 1)
    hm = jnp.where(t < T, h, 0.0)
    pooled = jnp.max(hm.reshape(B * N_OBJ, T_PAD, D_MODEL), axis=1, keepdims=True)
    pooledb = jnp.broadcast_to(pooled, (B * N_OBJ, T_PAD, D_MODEL))
    h2 = jnp.concatenate([h, pooledb.reshape(h.shape)], axis=1)
    h2 = _relu(jnp.dot(h2, w1[...], preferred_element_type=F32) + b1[...])
    h2 = _relu(jnp.dot(h2, w2[...], preferred_element_type=F32) + b2[...])
    h3 = _relu(jnp.dot(h2, w3[...], preferred_element_type=F32) + b3[...])
    out[...] = jnp.dot(h3, w4[...], preferred_element_type=F32) + b4[...]


def _map_pn_body(xin, w0, b0, w1, b1, w2, b2, wm0, bm0, wm1, bm1,
                 wo0, bo0, wo1, bo1, out):
    x = xin[0]
    h = _relu(jnp.dot(x, w0[...], preferred_element_type=F32) + b0[...])
    h = _relu(jnp.dot(h, w1[...], preferred_element_type=F32) + b1[...])
    h = _relu(jnp.dot(h, w2[...], preferred_element_type=F32) + b2[...])
    rid = lax.broadcasted_iota(jnp.int32, h.shape, 0)
    t = jnp.remainder(rid, P_PAD)
    hm = jnp.where(t < P_MAP, h, 0.0)
    pooled = jnp.max(hm.reshape(N_MAP, P_PAD, 64), axis=1, keepdims=True)
    pooledb = jnp.broadcast_to(pooled, (N_MAP, P_PAD, 64))
    h2 = jnp.concatenate([h, pooledb.reshape(h.shape)], axis=1)
    h2 = _relu(jnp.dot(h2, wm0[...], preferred_element_type=F32) + bm0[...])
    h2 = _relu(jnp.dot(h2, wm1[...], preferred_element_type=F32) + bm1[...])
    h2m = jnp.where(t < P_MAP, h2, 0.0)
    pooled2 = jnp.max(h2m.reshape(N_MAP, P_PAD, 64), axis=1)
    o = _relu(jnp.dot(pooled2, wo0[...], preferred_element_type=F32) + bo0[...])
    out[0] = jnp.dot(o, wo1[...], preferred_element_type=F32) + bo1[...]


def _prep_body(pos, posT, bias_ref, pe_ref):
    p = pos[0]
    pt = posT[0]
    xc = p[:, 0:1]
    yc = p[:, 1:2]
    zc = p[:, 2:3]
    dx = xc - pt[0:1, :]
    dy = yc - pt[1:2, :]
    dz = zc - pt[2:3, :]
    bias_ref[0] = dx * dx + dy * dy + dz * dz
    c = lax.broadcasted_iota(jnp.int32, (N_TOK, D_MODEL), 1)
    cc = c & 127
    j = (cc >> 1).astype(F32)
    invf = jnp.exp(j * (-math.log(10000.0) / 64.0))
    part = jnp.where(c < 128, yc, xc)
    val = part * (2.0 * math.pi) * invf
    pe_ref[0] = jnp.where((cc & 1) == 0, jnp.sin(val), jnp.cos(val))


_SC_WORKERS = 32
_SC_ROWS = (B * N_TOK) // _SC_WORKERS
_SC_CHUNKS = N_TOK // 16


def _sc_topk_body(d2_hbm, mask_hbm, row_v, mask_v):
    wid = lax.axis_index("s") * 2 + lax.axis_index("c")
    base = wid * _SC_ROWS
    zeros16 = jnp.zeros((16,), F32)
    ones16 = jnp.ones((16,), F32)

    def zero_body(i, carry):
        mask_v[pl.ds(i * 16, 16)] = zeros16
        return carry

    lax.fori_loop(0, _SC_CHUNKS, zero_body, 0)

    def row_body(r, carry):
        pltpu.sync_copy(d2_hbm.at[base + r], row_v)

        def chunk_body(c, cur):
            cur_d, cur_i = cur
            chunk = row_v[pl.ds(c * 16, 16)]
            idxv = lax.iota(jnp.int32, 16) + c * 16
            sd, si = plsc.sort_key_val(chunk, idxv)
            rd = lax.rev(cur_d, (0,))
            ri = lax.rev(cur_i, (0,))
            take = rd <= sd
            md = jnp.where(take, rd, sd)
            mi = jnp.where(take, ri, si)
            out = plsc.sort_key_val(md, mi)
            return (out[0], out[1])

        init = (jnp.full((16,), BIG, F32), jnp.zeros((16,), jnp.int32))
        _, cur_i = lax.fori_loop(0, _SC_CHUNKS, chunk_body, init)
        plsc.store_scatter(mask_v, [cur_i], ones16)
        pltpu.sync_copy(mask_v, mask_hbm.at[base + r])
        plsc.store_scatter(mask_v, [cur_i], zeros16)
        return carry

    lax.fori_loop(0, _SC_ROWS, row_body, 0)


def _sc_topk_mask(d2_flat):
    mesh = plsc.VectorSubcoreMesh(core_axis_name="c", subcore_axis_name="s")
    fn = functools.partial(
        pl.kernel,
        out_type=jax.ShapeDtypeStruct((B * N_TOK, N_TOK), F32),
        mesh=mesh,
        scratch_types=[pltpu.VMEM((N_TOK,), F32),
                       pltpu.VMEM((N_TOK,), F32)],
        compiler_params=pltpu.CompilerParams(needs_layout_passes=False),
    )(_sc_topk_body)
    return fn(d2_flat)


def _layer_body(x_ref, pe_ref, bias_ref, wq, bq, wk, bk, wv, bv, wo, bo,
                w1, bf1, w2, bf2, g1, be1, g2, be2, out_ref):
    x = x_ref[0]
    pe = pe_ref[0]
    mask01 = bias_ref[0].astype(jnp.bfloat16)
    qk = x + pe
    q = (_mm(qk, wq[...]) + bq[...]) * (DH ** -0.5)
    k = (_mm(qk, wk[...]) + bk[...]).astype(jnp.bfloat16)
    v = (_mm(x, wv[...]) + bv[...]).astype(jnp.bfloat16)
    qb = q.astype(jnp.bfloat16)
    ones8 = jnp.ones((N_TOK, 8), jnp.bfloat16)
    parts = []
    for h in range(N_HEADS):
        s = slice(DH * h, DH * (h + 1))
        S = lax.dot_general(qb[:, s], k[:, s], (((1,), (1,)), ((), ())),
                            preferred_element_type=F32)
        Eb = (jnp.exp(jnp.clip(S, -80.0, 60.0)) * mask01).astype(jnp.bfloat16)
        va = jnp.concatenate([v[:, s], ones8], axis=1)
        pv = jnp.dot(Eb, va, preferred_element_type=F32)
        parts.append(pv[:, :DH] / pv[:, DH:DH + 1])
    attn = jnp.concatenate(parts, axis=1)
    src = _ln(x + _mm(attn, wo[...]) + bo[...],
              g1[...], be1[...])
    ff = _mm(_relu(_mm(src, w1[...]) + bf1[...]), w2[...]) + bf2[...]
    out_ref[0] = _ln(src + ff, g2[...], be2[...])


def _wt(l):
    return l['W'].T


def _bb(l):
    return l['b'][None, :]


def kernel(obj_trajs, obj_trajs_mask, map_polylines, map_polylines_mask,
           obj_trajs_last_pos, obj_trajs_pos, map_polylines_center, params):
    p = params

    obj_in = jnp.concatenate(
        [obj_trajs, obj_trajs_mask[..., None].astype(F32)], axis=-1)
    obj_in = obj_in.reshape(B * N_OBJ, T, 30)
    obj_in = jnp.pad(obj_in, ((0, 0), (0, T_PAD - T), (0, 2)))
    obj_in = obj_in.reshape(B * N_OBJ * T_PAD, 32)
    w_pre = jnp.pad(_wt(p['agent_pre'][0]), ((0, 2), (0, 0)))
    obj_feat = pl.pallas_call(
        _obj_pn_body,
        out_shape=jax.ShapeDtypeStruct((B * N_OBJ * T_PAD, D_MODEL), F32),
    )(obj_in, w_pre, _bb(p['agent_pre'][0]),
      _wt(p['agent_mid'][0]), _bb(p['agent_mid'][0]),
      _wt(p['agent_mid'][1]), _bb(p['agent_mid'][1]),
      _wt(p['agent_out'][0]), _bb(p['agent_out'][0]),
      _wt(p['agent_out'][1]), _bb(p['agent_out'][1]))

    map_in = jnp.pad(map_polylines, ((0, 0), (0, 0), (0, P_PAD - P_MAP), (0, 7)))
    map_in = map_in.reshape(B, N_MAP * P_PAD, 16)
    w_mpre = jnp.pad(_wt(p['map_pre'][0]), ((0, 7), (0, 0)))
    full2 = lambda shp: pl.BlockSpec(shp, lambda b: (0, 0))
    map_feat = pl.pallas_call(
        _map_pn_body,
        grid=(B,),
        in_specs=[pl.BlockSpec((1, N_MAP * P_PAD, 16), lambda b: (b, 0, 0)),
                  full2((16, 64)), full2((1, 64)),
                  full2((64, 64)), full2((1, 64)),
                  full2((64, 64)), full2((1, 64)),
                  full2((128, 64)), full2((1, 64)),
                  full2((64, 64)), full2((1, 64)),
                  full2((64, 64)), full2((1, 64)),
                  full2((64, 256)), full2((1, 256))],
        out_specs=pl.BlockSpec((1, N_MAP, D_MODEL), lambda b: (b, 0, 0)),
        out_shape=jax.ShapeDtypeStruct((B, N_MAP, D_MODEL), F32),
    )(map_in, w_mpre, _bb(p['map_pre'][0]),
      _wt(p['map_pre'][1]), _bb(p['map_pre'][1]),
      _wt(p['map_pre'][2]), _bb(p['map_pre'][2]),
      _wt(p['map_mid'][0]), _bb(p['map_mid'][0]),
      _wt(p['map_mid'][1]), _bb(p['map_mid'][1]),
      _wt(p['map_out'][0]), _bb(p['map_out'][0]),
      _wt(p['map_out'][1]), _bb(p['map_out'][1]))

    obj_seq = obj_feat.reshape(B * N_OBJ, T_PAD, D_MODEL)[:, :T]
    obj_seq = obj_seq.reshape(B, N_OBJ * T, D_MODEL)
    tok = jnp.concatenate([obj_seq, map_feat], axis=1)
    pos = jnp.concatenate(
        [obj_trajs_pos.reshape(B, N_OBJ * T, 3), map_polylines_center], axis=1)
    pos = jnp.pad(pos, ((0, 0), (0, 0), (0, 5)))
    posT = jnp.swapaxes(pos, 1, 2)

    d2, pe = pl.pallas_call(
        _prep_body,
        grid=(B,),
        in_specs=[pl.BlockSpec((1, N_TOK, 8), lambda b: (b, 0, 0)),
                  pl.BlockSpec((1, 8, N_TOK), lambda b: (b, 0, 0))],
        out_specs=[pl.BlockSpec((1, N_TOK, N_TOK), lambda b: (b, 0, 0)),
                   pl.BlockSpec((1, N_TOK, D_MODEL), lambda b: (b, 0, 0))],
        out_shape=[jax.ShapeDtypeStruct((B, N_TOK, N_TOK), F32),
                   jax.ShapeDtypeStruct((B, N_TOK, D_MODEL), F32)],
    )(pos, posT)

    bias = _sc_topk_mask(d2.reshape(B * N_TOK, N_TOK)).reshape(B, N_TOK, N_TOK)

    x = tok
    row_spec = pl.BlockSpec((1, N_TOK, D_MODEL), lambda b: (b, 0, 0))
    layer_specs = [row_spec, row_spec,
                   pl.BlockSpec((1, N_TOK, N_TOK), lambda b: (b, 0, 0)),
                   full2((D_MODEL, D_MODEL)), full2((1, D_MODEL)),
                   full2((D_MODEL, D_MODEL)), full2((1, D_MODEL)),
                   full2((D_MODEL, D_MODEL)), full2((1, D_MODEL)),
                   full2((D_MODEL, D_MODEL)), full2((1, D_MODEL)),
                   full2((D_MODEL, 4 * D_MODEL)), full2((1, 4 * D_MODEL)),
                   full2((4 * D_MODEL, D_MODEL)), full2((1, D_MODEL)),
                   full2((1, D_MODEL)), full2((1, D_MODEL)),
                   full2((1, D_MODEL)), full2((1, D_MODEL))]
    for lp in p['layers']:
        x = pl.pallas_call(
            _layer_body,
            grid=(B,),
            in_specs=layer_specs,
            out_specs=row_spec,
            out_shape=jax.ShapeDtypeStruct((B, N_TOK, D_MODEL), F32),
        )(x, pe, bias,
          _wt(lp['q']), _bb(lp['q']), _wt(lp['k']), _bb(lp['k']),
          _wt(lp['v']), _bb(lp['v']), _wt(lp['o']), _bb(lp['o']),
          _wt(lp['ff1']), _bb(lp['ff1']), _wt(lp['ff2']), _bb(lp['ff2']),
          lp['ln1_g'][None, :], lp['ln1_b'][None, :],
          lp['ln2_g'][None, :], lp['ln2_b'][None, :])

    obj_out = x[:, :N_OBJ * T].reshape(B, N_OBJ, T, D_MODEL)
    map_out = jnp.broadcast_to(
        x[:, N_OBJ * T:][:, :, None, :], (B, N_MAP, T, D_MODEL))
    map_mask = map_polylines_mask.sum(axis=-1) > 0
    return obj_out, map_out, obj_trajs_mask, map_mask, obj_trajs_last_pos
